# baseline (speedup 1.0000x reference)
import jax
import jax.numpy as jnp
from jax import lax
from jax.experimental import pallas as pl
from jax.experimental.pallas import tpu as pltpu
from jax.experimental.pallas import tpu_sc as plsc

_B = 256
_NO = 1000
_NM = 64
_F = 8
_EC = 64

_ER = _NO * _EC
_S_HEAD = 48000
_BR1 = 8000
_G1 = _S_HEAD // _BR1
_BR2 = 12800
_G2 = _ER // _BR2
_ECH = 400
_OCH = 100

_N_EDGE = float(_B * _NO * _EC)

_NW = 32
_TAIL_START = _S_HEAD * _B
_TAIL_ELEMS = (_ER - _S_HEAD) * _B
_PER_W = _TAIL_ELEMS // _NW
_SC_CHUNK = 64000
_SC_CHUNKS = _PER_W // _SC_CHUNK


def _normalize_resident(x_ref, out_ref, count, chunk):
    n = count // chunk
    m = jnp.zeros((1, _F, _B), jnp.float32)
    for k in range(n):
        m = m + jnp.sum(x_ref[k * chunk:(k + 1) * chunk], axis=0,
                        keepdims=True)
    m = m * (1.0 / count)
    q = jnp.zeros((1, _F, _B), jnp.float32)
    for k in range(n):
        c = x_ref[k * chunk:(k + 1) * chunk] - m
        q = q + jnp.sum(c * c, axis=0, keepdims=True)
    inv = 1.0 / (jnp.sqrt(q * (1.0 / (count - 1))) + 1e-5)
    for k in range(n):
        sl = slice(k * chunk, (k + 1) * chunk)
        out_ref[sl] = (x_ref[sl] - m) * inv


def _stats_opes_kernel(edge_ref, opes_ref, mas_ref,
                       opes_out_ref, mas_out_ref, stats_ref):
    step = pl.program_id(0)

    @pl.when(step == 0)
    def _init():
        stats_ref[0] = 0.0
        stats_ref[1] = 0.0

    s = jnp.float32(0.0)
    q = jnp.float32(0.0)
    for k in range(_BR1 // _ECH):
        c = edge_ref[k * _ECH:(k + 1) * _ECH, :]
        s = s + jnp.sum(c)
        q = q + jnp.sum(c * c)
    stats_ref[0] += s
    stats_ref[1] += q

    @pl.when(step == 0)
    def _features():
        _normalize_resident(opes_ref, opes_out_ref, _NO, _OCH)
        _normalize_resident(mas_ref, mas_out_ref, _NM, _NM)


def _sc_tail_stats(flat_ref, s_out_ref, q_out_ref, buf, sacc, qacc):
    wid = lax.axis_index("s") * 2 + lax.axis_index("c")
    base = _TAIL_START + wid * _PER_W
    z = jnp.zeros((16,), jnp.float32)
    s0, q0, s1, q1 = z, z, z, z
    for ch in range(_SC_CHUNKS):
        pltpu.sync_copy(flat_ref.at[pl.ds(base + ch * _SC_CHUNK, _SC_CHUNK)],
                        buf)

        def body(j, carry):
            cs0, cq0, cs1, cq1 = carry
            b0 = j * 128
            for u in range(4):
                v = buf[pl.ds(b0 + u * 32, 16)]
                w = buf[pl.ds(b0 + u * 32 + 16, 16)]
                cs0 = cs0 + v
                cq0 = cq0 + v * v
                cs1 = cs1 + w
                cq1 = cq1 + w * w
            return (cs0, cq0, cs1, cq1)

        s0, q0, s1, q1 = lax.fori_loop(0, _SC_CHUNK // 128, body,
                                       (s0, q0, s1, q1))
    sacc[...] = s0 + s1
    qacc[...] = q0 + q1
    pltpu.sync_copy(sacc, s_out_ref.at[wid])
    pltpu.sync_copy(qacc, q_out_ref.at[wid])


def _edge_norm_kernel(stats_ref, edge_ref, edge_out_ref):
    s = stats_ref[0]
    q = stats_ref[1]
    gm = s / _N_EDGE
    var = (q - _N_EDGE * gm * gm) * (1.0 / (_N_EDGE - 1.0))
    a = 1.0 / (jnp.sqrt(var) + 1e-5)
    b = -gm * a
    for k in range(_BR2 // _ECH):
        sl = slice(k * _ECH, (k + 1) * _ECH)
        edge_out_ref[sl, :] = edge_ref[sl, :] * a + b


@jax.jit
def kernel(batch_opr_features, batch_station_features, batch_edge_features):
    edge_t = jnp.transpose(batch_edge_features, (1, 2, 0)).reshape(_ER, _B)
    opes_t = jnp.transpose(batch_opr_features, (1, 2, 0))
    mas_t = jnp.transpose(batch_station_features, (1, 2, 0))
    edge_flat = edge_t.reshape(-1)

    opes_out, mas_out, stats_head = pl.pallas_call(
        _stats_opes_kernel,
        grid=(_G1,),
        in_specs=[
            pl.BlockSpec((_BR1, _B), lambda i: (i, 0)),
            pl.BlockSpec((_NO, _F, _B), lambda i: (0, 0, 0)),
            pl.BlockSpec((_NM, _F, _B), lambda i: (0, 0, 0)),
        ],
        out_specs=[
            pl.BlockSpec((_NO, _F, _B), lambda i: (0, 0, 0)),
            pl.BlockSpec((_NM, _F, _B), lambda i: (0, 0, 0)),
            pl.BlockSpec(memory_space=pltpu.SMEM),
        ],
        out_shape=[
            jax.ShapeDtypeStruct((_NO, _F, _B), jnp.float32),
            jax.ShapeDtypeStruct((_NM, _F, _B), jnp.float32),
            jax.ShapeDtypeStruct((2,), jnp.float32),
        ],
        compiler_params=pltpu.CompilerParams(
            dimension_semantics=("arbitrary",),
        ),
    )(edge_t, opes_t, mas_t)

    sc_mesh = plsc.VectorSubcoreMesh(core_axis_name="c", subcore_axis_name="s")
    s_rows, q_rows = pl.kernel(
        _sc_tail_stats,
        out_type=[
            jax.ShapeDtypeStruct((_NW, 16), jnp.float32),
            jax.ShapeDtypeStruct((_NW, 16), jnp.float32),
        ],
        mesh=sc_mesh,
        scratch_types=[
            pltpu.VMEM((_SC_CHUNK,), jnp.float32),
            pltpu.VMEM((16,), jnp.float32),
            pltpu.VMEM((16,), jnp.float32),
        ],
    )(edge_flat)

    stats = stats_head + jnp.stack([jnp.sum(s_rows), jnp.sum(q_rows)])

    edge_out = pl.pallas_call(
        _edge_norm_kernel,
        grid=(_G2,),
        in_specs=[
            pl.BlockSpec(memory_space=pltpu.SMEM),
            pl.BlockSpec((_BR2, _B), lambda i: (i, 0)),
        ],
        out_specs=pl.BlockSpec((_BR2, _B), lambda i: (i, 0)),
        out_shape=jax.ShapeDtypeStruct((_ER, _B), jnp.float32),
        compiler_params=pltpu.CompilerParams(
            dimension_semantics=("arbitrary",),
        ),
    )(stats, edge_t)

    return (
        jnp.transpose(opes_out, (2, 0, 1)),
        jnp.transpose(mas_out, (2, 0, 1)),
        jnp.transpose(edge_out.reshape(_NO, _EC, _B), (2, 0, 1)),
    )


# speedup vs baseline: 1.6243x; 1.6243x over previous
import jax
import jax.numpy as jnp
from jax import lax
from jax.experimental import pallas as pl
from jax.experimental.pallas import tpu as pltpu
from jax.experimental.pallas import tpu_sc as plsc

_B = 256
_NO = 1000
_NM = 64
_F = 8
_EC = 64

_ER = _NO * _EC
_S_HEAD = 48128
_BR1 = 6016
_G1 = _S_HEAD // _BR1
_BR2 = 12800
_G2 = _ER // _BR2
_ECH1 = 376
_ECH2 = 400
_OCH = 100

_N_EDGE = float(_B * _NO * _EC)

_NW = 32
_PER_W_ROWS = (_ER - _S_HEAD) // _NW
_SC_ROWS = 248
_SC_CHUNKS = _PER_W_ROWS // _SC_ROWS


def _normalize_resident(x_ref, out_ref, count, chunk):
    n = count // chunk
    m = jnp.zeros((1, _F, _B), jnp.float32)
    for k in range(n):
        m = m + jnp.sum(x_ref[k * chunk:(k + 1) * chunk], axis=0,
                        keepdims=True)
    m = m * (1.0 / count)
    q = jnp.zeros((1, _F, _B), jnp.float32)
    for k in range(n):
        c = x_ref[k * chunk:(k + 1) * chunk] - m
        q = q + jnp.sum(c * c, axis=0, keepdims=True)
    inv = 1.0 / (jnp.sqrt(q * (1.0 / (count - 1))) + 1e-5)
    for k in range(n):
        sl = slice(k * chunk, (k + 1) * chunk)
        out_ref[sl] = (x_ref[sl] - m) * inv


def _stats_opes_kernel(edge_ref, opes_ref, mas_ref,
                       opes_out_ref, mas_out_ref, stats_ref):
    step = pl.program_id(0)

    @pl.when(step == 0)
    def _init():
        stats_ref[0] = 0.0
        stats_ref[1] = 0.0

    s = jnp.float32(0.0)
    q = jnp.float32(0.0)
    for k in range(_BR1 // _ECH1):
        c = edge_ref[k * _ECH1:(k + 1) * _ECH1, :]
        s = s + jnp.sum(c)
        q = q + jnp.sum(c * c)
    stats_ref[0] += s
    stats_ref[1] += q

    @pl.when(step == 0)
    def _features():
        _normalize_resident(opes_ref, opes_out_ref, _NO, _OCH)
        _normalize_resident(mas_ref, mas_out_ref, _NM, _NM)


def _sc_tail_stats(edge_ref, s_out_ref, q_out_ref, buf, sacc, qacc):
    wid = lax.axis_index("s") * 2 + lax.axis_index("c")
    row_base = _S_HEAD + wid * _PER_W_ROWS
    z = jnp.zeros((16,), jnp.float32)
    s0, q0, s1, q1 = z, z, z, z
    for ch in range(_SC_CHUNKS):
        pltpu.sync_copy(
            edge_ref.at[pl.ds(row_base + ch * _SC_ROWS, _SC_ROWS)], buf)

        def body(r, carry):
            cs0, cq0, cs1, cq1 = carry
            for u in range(8):
                v = buf[r, pl.ds(u * 32, 16)]
                w = buf[r, pl.ds(u * 32 + 16, 16)]
                cs0 = cs0 + v
                cq0 = cq0 + v * v
                cs1 = cs1 + w
                cq1 = cq1 + w * w
            return (cs0, cq0, cs1, cq1)

        s0, q0, s1, q1 = lax.fori_loop(0, _SC_ROWS, body, (s0, q0, s1, q1))
    sacc[...] = s0 + s1
    qacc[...] = q0 + q1
    pltpu.sync_copy(sacc, s_out_ref.at[wid])
    pltpu.sync_copy(qacc, q_out_ref.at[wid])


def _edge_norm_kernel(stats_ref, edge_ref, edge_out_ref):
    s = stats_ref[0]
    q = stats_ref[1]
    gm = s / _N_EDGE
    var = (q - _N_EDGE * gm * gm) * (1.0 / (_N_EDGE - 1.0))
    a = 1.0 / (jnp.sqrt(var) + 1e-5)
    b = -gm * a
    for k in range(_BR2 // _ECH2):
        sl = slice(k * _ECH2, (k + 1) * _ECH2)
        edge_out_ref[sl, :] = edge_ref[sl, :] * a + b


@jax.jit
def kernel(batch_opr_features, batch_station_features, batch_edge_features):
    edge_t = jnp.transpose(batch_edge_features, (1, 2, 0)).reshape(_ER, _B)
    opes_t = jnp.transpose(batch_opr_features, (1, 2, 0))
    mas_t = jnp.transpose(batch_station_features, (1, 2, 0))

    opes_out, mas_out, stats_head = pl.pallas_call(
        _stats_opes_kernel,
        grid=(_G1,),
        in_specs=[
            pl.BlockSpec((_BR1, _B), lambda i: (i, 0)),
            pl.BlockSpec((_NO, _F, _B), lambda i: (0, 0, 0)),
            pl.BlockSpec((_NM, _F, _B), lambda i: (0, 0, 0)),
        ],
        out_specs=[
            pl.BlockSpec((_NO, _F, _B), lambda i: (0, 0, 0)),
            pl.BlockSpec((_NM, _F, _B), lambda i: (0, 0, 0)),
            pl.BlockSpec(memory_space=pltpu.SMEM),
        ],
        out_shape=[
            jax.ShapeDtypeStruct((_NO, _F, _B), jnp.float32),
            jax.ShapeDtypeStruct((_NM, _F, _B), jnp.float32),
            jax.ShapeDtypeStruct((2,), jnp.float32),
        ],
        compiler_params=pltpu.CompilerParams(
            dimension_semantics=("arbitrary",),
        ),
    )(edge_t, opes_t, mas_t)

    sc_mesh = plsc.VectorSubcoreMesh(core_axis_name="c", subcore_axis_name="s")
    s_rows, q_rows = pl.kernel(
        _sc_tail_stats,
        out_type=[
            jax.ShapeDtypeStruct((_NW, 16), jnp.float32),
            jax.ShapeDtypeStruct((_NW, 16), jnp.float32),
        ],
        mesh=sc_mesh,
        scratch_types=[
            pltpu.VMEM((_SC_ROWS, _B), jnp.float32),
            pltpu.VMEM((16,), jnp.float32),
            pltpu.VMEM((16,), jnp.float32),
        ],
    )(edge_t)

    stats = stats_head + jnp.stack([jnp.sum(s_rows), jnp.sum(q_rows)])

    edge_out = pl.pallas_call(
        _edge_norm_kernel,
        grid=(_G2,),
        in_specs=[
            pl.BlockSpec(memory_space=pltpu.SMEM),
            pl.BlockSpec((_BR2, _B), lambda i: (i, 0)),
        ],
        out_specs=pl.BlockSpec((_BR2, _B), lambda i: (i, 0)),
        out_shape=jax.ShapeDtypeStruct((_ER, _B), jnp.float32),
        compiler_params=pltpu.CompilerParams(
            dimension_semantics=("arbitrary",),
        ),
    )(stats, edge_t)

    return (
        jnp.transpose(opes_out, (2, 0, 1)),
        jnp.transpose(mas_out, (2, 0, 1)),
        jnp.transpose(edge_out.reshape(_NO, _EC, _B), (2, 0, 1)),
    )


# speedup vs baseline: 2.0309x; 1.2503x over previous
import jax
import jax.numpy as jnp
from jax.experimental import pallas as pl
from jax.experimental.pallas import tpu as pltpu

_B = 256
_NO = 1000
_NM = 64
_F = 8
_EC = 64

_ER = _NO * _EC
_BR = 12800
_GRID = _ER // _BR
_ECH = 400
_OCH = 100

_N_EDGE = float(_B * _NO * _EC)


def _normalize_resident(x_ref, out_ref, count, chunk):
    n = count // chunk
    m = jnp.zeros((1, _F, _B), jnp.float32)
    for k in range(n):
        m = m + jnp.sum(x_ref[k * chunk:(k + 1) * chunk], axis=0,
                        keepdims=True)
    m = m * (1.0 / count)
    q = jnp.zeros((1, _F, _B), jnp.float32)
    for k in range(n):
        c = x_ref[k * chunk:(k + 1) * chunk] - m
        q = q + jnp.sum(c * c, axis=0, keepdims=True)
    inv = 1.0 / (jnp.sqrt(q * (1.0 / (count - 1))) + 1e-5)
    for k in range(n):
        sl = slice(k * chunk, (k + 1) * chunk)
        out_ref[sl] = (x_ref[sl] - m) * inv


def _stats_opes_kernel(edge_ref, opes_ref, mas_ref,
                       opes_out_ref, mas_out_ref, stats_ref):
    step = pl.program_id(0)

    @pl.when(step == 0)
    def _init():
        stats_ref[0] = 0.0
        stats_ref[1] = 0.0

    s = jnp.float32(0.0)
    q = jnp.float32(0.0)
    for k in range(_BR // _ECH):
        c = edge_ref[k * _ECH:(k + 1) * _ECH, :]
        s = s + jnp.sum(c)
        q = q + jnp.sum(c * c)
    stats_ref[0] += s
    stats_ref[1] += q

    @pl.when(step == 0)
    def _features():
        _normalize_resident(opes_ref, opes_out_ref, _NO, _OCH)
        _normalize_resident(mas_ref, mas_out_ref, _NM, _NM)


def _edge_norm_kernel(stats_ref, edge_ref, edge_out_ref):
    s = stats_ref[0]
    q = stats_ref[1]
    gm = s / _N_EDGE
    var = (q - _N_EDGE * gm * gm) * (1.0 / (_N_EDGE - 1.0))
    a = 1.0 / (jnp.sqrt(var) + 1e-5)
    b = -gm * a
    for k in range(_BR // _ECH):
        sl = slice(k * _ECH, (k + 1) * _ECH)
        edge_out_ref[sl, :] = edge_ref[sl, :] * a + b


@jax.jit
def kernel(batch_opr_features, batch_station_features, batch_edge_features):
    edge_t = jnp.transpose(batch_edge_features, (1, 2, 0)).reshape(_ER, _B)
    opes_t = jnp.transpose(batch_opr_features, (1, 2, 0))
    mas_t = jnp.transpose(batch_station_features, (1, 2, 0))

    opes_out, mas_out, stats = pl.pallas_call(
        _stats_opes_kernel,
        grid=(_GRID,),
        in_specs=[
            pl.BlockSpec((_BR, _B), lambda i: (i, 0)),
            pl.BlockSpec((_NO, _F, _B), lambda i: (0, 0, 0)),
            pl.BlockSpec((_NM, _F, _B), lambda i: (0, 0, 0)),
        ],
        out_specs=[
            pl.BlockSpec((_NO, _F, _B), lambda i: (0, 0, 0)),
            pl.BlockSpec((_NM, _F, _B), lambda i: (0, 0, 0)),
            pl.BlockSpec(memory_space=pltpu.SMEM),
        ],
        out_shape=[
            jax.ShapeDtypeStruct((_NO, _F, _B), jnp.float32),
            jax.ShapeDtypeStruct((_NM, _F, _B), jnp.float32),
            jax.ShapeDtypeStruct((2,), jnp.float32),
        ],
        compiler_params=pltpu.CompilerParams(
            dimension_semantics=("arbitrary",),
        ),
    )(edge_t, opes_t, mas_t)

    edge_out = pl.pallas_call(
        _edge_norm_kernel,
        grid=(_GRID,),
        in_specs=[
            pl.BlockSpec(memory_space=pltpu.SMEM),
            pl.BlockSpec((_BR, _B), lambda i: (i, 0)),
        ],
        out_specs=pl.BlockSpec((_BR, _B), lambda i: (i, 0)),
        out_shape=jax.ShapeDtypeStruct((_ER, _B), jnp.float32),
        compiler_params=pltpu.CompilerParams(
            dimension_semantics=("arbitrary",),
        ),
    )(stats, edge_t)

    return (
        jnp.transpose(opes_out, (2, 0, 1)),
        jnp.transpose(mas_out, (2, 0, 1)),
        jnp.transpose(edge_out.reshape(_NO, _EC, _B), (2, 0, 1)),
    )
